# baseline (speedup 1.0000x reference)
import functools

import jax
import jax.numpy as jnp
from jax import lax
from jax.experimental import pallas as pl
from jax.experimental.pallas import tpu as pltpu
from jax.experimental.pallas import tpu_sc as plsc

N, NP = 10000, 10240
D_IN, H, B, A = 256, 1024, 64, 64
E, ET = 160000, 170000
NS, NC = 16, 2
CW = 128
NCH = H // CW
BM = 1024
NPC = NP // NC
BR = 64
EB2 = 88
EPT2 = EB2 * BR
CAP = EPT2 * NS
RPT = NP // NS
RPC = NPC // NS


def _mm_body(has_ln, x_ref, w_ref, as_ref, ad_ref, *rest):
    if has_ln:
        b_ref, g_ref, nb_ref, ht_ref, ssrc_ref, sdst_ref, mb_ref, mx_sc = rest
        xb = jnp.concatenate([x_ref[c] for c in range(NCH)], axis=-1)
        xb = xb + b_ref[...]
        mu = jnp.mean(xb, axis=-1, keepdims=True)
        var = jnp.mean((xb - mu) ** 2, axis=-1, keepdims=True)
        xb = (xb - mu) / jnp.sqrt(var + 1e-5) * g_ref[...] + nb_ref[...]
        xb = jnp.maximum(xb, 0.0)
    else:
        ht_ref, ssrc_ref, sdst_ref, mb_ref, mx_sc = rest
        xb = x_ref[...]
    h = jnp.dot(xb, w_ref[...], preferred_element_type=jnp.float32)
    for c in range(NCH):
        ht_ref[c] = h[:, c * CW:(c + 1) * CW]
    s1 = jnp.dot(h, as_ref[...], preferred_element_type=jnp.float32)
    s2 = jnp.dot(h, ad_ref[...], preferred_element_type=jnp.float32)
    ssrc_ref[...] = s1
    sdst_ref[...] = s2

    i = pl.program_id(0)

    @pl.when(i == 0)
    def _():
        mx_sc[0] = -3e38
        mx_sc[1] = -3e38
    mx_sc[0] = jnp.maximum(mx_sc[0], jnp.max(s1))
    mx_sc[1] = jnp.maximum(mx_sc[1], jnp.max(s2))
    msum = mx_sc[0] + mx_sc[1]
    mb_ref[...] = jnp.full((1, 16), jnp.where(msum >= 0.0, msum, 0.2 * msum),
                           jnp.float32)


def _tc_layer(x, W, a_s, a_d, ln_params):
    nblk = NP // BM
    if x.ndim == 3:
        xspec = pl.BlockSpec((NCH, BM, CW), lambda i: (0, i, 0))
        K = H
    else:
        K = x.shape[1]
        xspec = pl.BlockSpec((BM, K), lambda i: (i, 0))
    args = [x, W, a_s.reshape(H, 1), a_d.reshape(H, 1)]
    in_specs = [
        xspec,
        pl.BlockSpec((K, H), lambda i: (0, 0)),
        pl.BlockSpec((H, 1), lambda i: (0, 0)),
        pl.BlockSpec((H, 1), lambda i: (0, 0)),
    ]
    if ln_params is not None:
        bp, g, nb = ln_params
        args += [bp.reshape(1, H), g.reshape(1, H), nb.reshape(1, H)]
        in_specs += [pl.BlockSpec((1, H), lambda i: (0, 0))] * 3
    ht, ssrc, sdst, mb = pl.pallas_call(
        functools.partial(_mm_body, ln_params is not None),
        grid=(nblk,),
        in_specs=in_specs,
        out_specs=[
            pl.BlockSpec((NCH, BM, CW), lambda i: (0, i, 0)),
            pl.BlockSpec((BM, 1), lambda i: (i, 0)),
            pl.BlockSpec((BM, 1), lambda i: (i, 0)),
            pl.BlockSpec((1, 16), lambda i: (0, 0)),
        ],
        out_shape=[
            jax.ShapeDtypeStruct((NCH, NP, CW), jnp.float32),
            jax.ShapeDtypeStruct((NP, 1), jnp.float32),
            jax.ShapeDtypeStruct((NP, 1), jnp.float32),
            jax.ShapeDtypeStruct((1, 16), jnp.float32),
        ],
        scratch_shapes=[pltpu.SMEM((2,), jnp.float32)],
    )(*args)
    return (ht.reshape(NCH * NP, CW), ssrc.reshape(NP), sdst.reshape(NP),
            mb.reshape(16))


def _sc_gat(htab, ssrc, sdst, mb, src4, dst4, nreal2):
    mesh = plsc.VectorSubcoreMesh(
        core_axis_name="c", subcore_axis_name="s",
        num_cores=NC, num_subcores=NS)

    @functools.partial(
        pl.kernel, mesh=mesh,
        compiler_params=pltpu.CompilerParams(
            needs_layout_passes=False, use_tc_tiling_on_sc=False),
        out_type=jax.ShapeDtypeStruct((NCH, NP, CW), jnp.float32),
        scratch_types=[
            pltpu.VMEM((NP,), jnp.float32),
            pltpu.VMEM((NP,), jnp.float32),
            pltpu.VMEM((NP,), jnp.float32),
            pltpu.VMEM((EB2, BR), jnp.int32),
            pltpu.VMEM((EB2, BR), jnp.int32),
            pltpu.VMEM((EB2, BR), jnp.int32),
            pltpu.VMEM((EB2, BR), jnp.float32),
            pltpu.VMEM((2, BR), jnp.int32),
            pltpu.VMEM((2, BR, CW), jnp.float32),
            pltpu.VMEM((BR, CW), jnp.float32),
            pltpu.VMEM((RPT,), jnp.float32),
            pltpu.VMEM((RPT,), jnp.float32),
            pltpu.HBM((NC, NS, NP), jnp.float32),
            pltpu.VMEM_SHARED((NP,), jnp.float32),
            pltpu.VMEM_SHARED((NPC, CW), jnp.float32),
            pltpu.VMEM((16,), jnp.float32),
            pltpu.VMEM((16,), jnp.int32),
            pltpu.SemaphoreType.DMA,
            pltpu.SemaphoreType.DMA,
        ],
    )
    def k(htab_h, ssrc_h, sdst_h, mb_h, src_h, dst_h, nreal_h, agg_h,
          ssrc_t, sdst_t, s_t, src_t, dst_t, dstl_t, alpha_t, idx2, rowbuf2,
          zbuf, tmp_t, acc_t, s_all, s_acc, out_chunk, mb_t, nreal_t,
          gsem, ssem):
        cid = lax.axis_index("c")
        sid = lax.axis_index("s")
        pltpu.sync_copy(ssrc_h, ssrc_t)
        pltpu.sync_copy(sdst_h, sdst_t)
        pltpu.sync_copy(mb_h, mb_t)
        pltpu.sync_copy(nreal_h.at[cid], nreal_t)
        pltpu.sync_copy(src_h.at[cid, sid], src_t)
        pltpu.sync_copy(dst_h.at[cid, sid], dst_t)

        zf = jnp.zeros((16,), jnp.float32)

        def zero_s(i, c):
            s_t[pl.ds(i * 16, 16)] = zf
            return c
        lax.fori_loop(0, NP // 16, zero_s, 0)

        def zero_z(r, c):
            for q in range(CW // 16):
                zbuf[r, pl.ds(q * 16, 16)] = zf
            return c
        lax.fori_loop(0, BR, zero_z, 0)

        cbase = cid * NPC

        def loc_body(b, c):
            for kq in range(BR // 16):
                sl = pl.ds(kq * 16, 16)
                dstl_t[b, sl] = dst_t[b, sl] - cbase
            return c
        lax.fori_loop(0, EB2, loc_body, 0)

        mb = mb_t[pl.ds(0, 16)]
        nv = nreal_t[pl.ds(0, 16)]

        ebase = sid * EPT2
        iota16 = lax.iota(jnp.int32, 16)

        def e_body(b, c):
            for kq in range(BR // 16):
                sl = pl.ds(kq * 16, 16)
                sv = src_t[b, sl]
                dv = dst_t[b, sl]
                e = (plsc.load_gather(ssrc_t, [sv])
                     + plsc.load_gather(sdst_t, [dv]))
                e = jnp.where(e >= 0.0, e, 0.2 * e) - mb
                ex = jnp.exp(e)
                gidx = ebase + b * BR + kq * 16 + iota16
                ex = jnp.where(gidx < nv, ex, 0.0)
                alpha_t[b, sl] = ex
                plsc.addupdate_scatter(s_t, [dv], ex)
            return c
        lax.fori_loop(0, EB2, e_body, 0)

        plsc.subcore_barrier()
        pltpu.sync_copy(s_t, s_all.at[cid, sid])
        plsc.subcore_barrier()
        base = sid * RPT
        pltpu.sync_copy(s_all.at[cid, 0, pl.ds(base, RPT)], acc_t)

        def t_body(t, c):
            pltpu.sync_copy(s_all.at[cid, t, pl.ds(base, RPT)], tmp_t)

            def add_body(j, c2):
                sl = pl.ds(j * 16, 16)
                acc_t[sl] = acc_t[sl] + tmp_t[sl]
                return c2
            lax.fori_loop(0, RPT // 16, add_body, 0)
            return c
        lax.fori_loop(1, NS, t_body, 0)
        pltpu.sync_copy(acc_t, s_acc.at[pl.ds(base, RPT)])
        plsc.subcore_barrier()
        pltpu.sync_copy(s_acc, s_t)

        def a_body(b, c):
            for kq in range(BR // 16):
                sl = pl.ds(kq * 16, 16)
                dv = dst_t[b, sl]
                sden = plsc.load_gather(s_t, [dv])
                alpha_t[b, sl] = alpha_t[b, sl] / sden
            return c
        lax.fori_loop(0, EB2, a_body, 0)

        for cc in range(NCH):
            goff = cc * NP
            plsc.subcore_barrier()
            for z in range(RPC // BR):
                pltpu.sync_copy(zbuf, out_chunk.at[pl.ds(sid * RPC + z * BR, BR), :])
            plsc.subcore_barrier()

            def build_idx(q, b):
                for kq in range(BR // 16):
                    sl = pl.ds(kq * 16, 16)
                    idx2[q, sl] = src_t[b, sl] + goff

            build_idx(0, 0)
            pltpu.async_copy(htab_h.at[idx2.at[0]], rowbuf2.at[0], gsem)

            def grp(b, c):
                p = b % 2

                @pl.when(b >= 1)
                def _():
                    pltpu.make_async_copy(
                        rowbuf2.at[1 - p], out_chunk.at[dstl_t.at[b - 1]],
                        ssem).wait()

                @pl.when(b + 1 < EB2)
                def _():
                    build_idx(1 - p, b + 1)
                    pltpu.async_copy(htab_h.at[idx2.at[1 - p]],
                                     rowbuf2.at[1 - p], gsem)

                pltpu.make_async_copy(htab_h.at[idx2.at[p]], rowbuf2.at[p],
                                      gsem).wait()

                @plsc.parallel_loop(0, BR // 16, unroll=2)
                def _(kq2):
                    av16 = alpha_t[b, pl.ds(kq2 * 16, 16)]
                    for u in range(16):
                        av = jnp.full((16,), av16[u], jnp.float32)
                        r = kq2 * 16 + u
                        for q in range(CW // 16):
                            ql = pl.ds(q * 16, 16)
                            rowbuf2[p, r, ql] = rowbuf2[p, r, ql] * av

                pltpu.async_copy(rowbuf2.at[p], out_chunk.at[dstl_t.at[b]],
                                 ssem, add=True)
                return c
            lax.fori_loop(0, EB2, grp, 0)

            pltpu.make_async_copy(rowbuf2.at[(EB2 - 1) % 2],
                                  out_chunk.at[dstl_t.at[EB2 - 1]], ssem).wait()

            plsc.subcore_barrier()
            for z in range(RPC // BR):
                r0 = sid * RPC + z * BR
                pltpu.sync_copy(out_chunk.at[pl.ds(r0, BR), :],
                                agg_h.at[cc, pl.ds(cbase + r0, BR), :])

    return k(htab, ssrc, sdst, mb, src4, dst4, nreal2)


def _pool_body(x_ref, b_ref, g_ref, nb_ref, bat_ref, wg_ref, bg_ref,
               pn_ref, sp_ref):
    i = pl.program_id(0)
    xb = jnp.concatenate([x_ref[c] for c in range(NCH)], axis=-1)
    xb = xb + b_ref[...]
    mu = jnp.mean(xb, axis=-1, keepdims=True)
    var = jnp.mean((xb - mu) ** 2, axis=-1, keepdims=True)
    xb = (xb - mu) / jnp.sqrt(var + 1e-5) * g_ref[...] + nb_ref[...]
    xb = jnp.maximum(xb, 0.0)
    gl = jnp.dot(xb, wg_ref[...], preferred_element_type=jnp.float32) + bg_ref[...]
    ex = jnp.exp(jax.nn.sigmoid(gl))
    onehot = (bat_ref[...] == lax.broadcasted_iota(jnp.int32, (BM, B), 1)
              ).astype(jnp.float32)
    w = onehot * ex
    pp = lax.dot_general(w, xb, (((0,), (0,)), ((), ())),
                         preferred_element_type=jnp.float32)
    sp1 = lax.dot_general(w, jnp.ones((BM, 1), jnp.float32),
                          (((0,), (0,)), ((), ())),
                          preferred_element_type=jnp.float32)

    @pl.when(i == 0)
    def _():
        pn_ref[...] = jnp.zeros_like(pn_ref)
        sp_ref[...] = jnp.zeros_like(sp_ref)
    pn_ref[...] += pp
    sp_ref[...] += sp1


def _tc_pool(x, ln_params, bat2, Wg, bg):
    bp, g, nb = ln_params
    nblk = NP // BM
    return pl.pallas_call(
        _pool_body,
        grid=(nblk,),
        in_specs=[
            pl.BlockSpec((NCH, BM, CW), lambda i: (0, i, 0)),
            pl.BlockSpec((1, H), lambda i: (0, 0)),
            pl.BlockSpec((1, H), lambda i: (0, 0)),
            pl.BlockSpec((1, H), lambda i: (0, 0)),
            pl.BlockSpec((BM, 1), lambda i: (i, 0)),
            pl.BlockSpec((H, 1), lambda i: (0, 0)),
            pl.BlockSpec((1, 1), lambda i: (0, 0)),
        ],
        out_specs=[
            pl.BlockSpec((B, H), lambda i: (0, 0)),
            pl.BlockSpec((B, 1), lambda i: (0, 0)),
        ],
        out_shape=[
            jax.ShapeDtypeStruct((B, H), jnp.float32),
            jax.ShapeDtypeStruct((B, 1), jnp.float32),
        ],
    )(x, bp.reshape(1, H), g.reshape(1, H), nb.reshape(1, H),
      bat2, Wg, bg.reshape(1, 1))


def _mlp_body(pn_ref, sp_ref, *refs):
    y = pn_ref[...] / (sp_ref[...] + 1e-16)
    for l in range(5):
        fw, fb, fg, fbeta = refs[4 * l:4 * l + 4]
        y = jnp.dot(y, fw[...], preferred_element_type=jnp.float32) + fb[...]
        mu = jnp.mean(y, axis=-1, keepdims=True)
        var = jnp.mean((y - mu) ** 2, axis=-1, keepdims=True)
        y = (y - mu) / jnp.sqrt(var + 1e-5) * fg[...] + fbeta[...]
        y = jnp.maximum(y, 0.0)
    wo, bo, out_ref = refs[20], refs[21], refs[22]
    out_ref[...] = (jnp.dot(y, wo[...], preferred_element_type=jnp.float32)
                    + bo[...])


def _tc_mlp(pn, sp, params):
    args = [pn, sp]
    for l in range(5):
        args += [params[f"fW{l}"], params[f"fb{l}"].reshape(1, H),
                 params[f"fg{l}"].reshape(1, H), params[f"fbeta{l}"].reshape(1, H)]
    args += [params["Wo"], params["bo"].reshape(1, A)]
    return pl.pallas_call(
        _mlp_body,
        out_shape=jax.ShapeDtypeStruct((B, A), jnp.float32),
    )(*args)


def kernel(tree_x, edge_index, batch, params):
    idt = edge_index.dtype
    loops = jnp.arange(N, dtype=idt)
    srcs = jnp.concatenate([edge_index[0], loops])
    dsts = jnp.concatenate([edge_index[1], loops])
    half = (dsts >= NPC).astype(jnp.int32)
    pos0 = jnp.cumsum(1 - half) - 1
    pos1 = jnp.cumsum(half) - 1
    pos = jnp.where(half == 0, pos0, CAP + pos1)
    src2 = jnp.zeros((NC * CAP,), idt).at[pos].set(srcs)
    dst_init = jnp.concatenate(
        [jnp.zeros((CAP,), idt), jnp.full((CAP,), NPC, idt)])
    dst2 = dst_init.at[pos].set(dsts)
    n0 = jnp.sum(1 - half).astype(jnp.int32)
    nreal2 = jnp.stack([jnp.full((16,), 1, jnp.int32) * n0,
                        jnp.full((16,), 1, jnp.int32) * (ET - n0)])
    src4 = src2.reshape(NC, NS, EB2, BR)
    dst4 = dst2.reshape(NC, NS, EB2, BR)
    x = jnp.zeros((NP, D_IN), jnp.float32).at[:N].set(tree_x)
    bat2 = jnp.full((NP, 1), B, jnp.int32).at[:N, 0].set(batch)

    ln = None
    for i in range(5):
        ht, ssrc, sdst, mb = _tc_layer(
            x, params[f"W{i}"], params[f"as{i}"], params[f"ad{i}"], ln)
        x = _sc_gat(ht, ssrc, sdst, mb, src4, dst4, nreal2)
        ln = (params[f"b{i}"], params[f"ng{i}"], params[f"nb{i}"])

    pn, sp = _tc_pool(x, ln, bat2, params["Wg"], params["bg"])
    return _tc_mlp(pn, sp, params)


# speedup vs baseline: 2.6250x; 2.6250x over previous
import functools

import jax
import jax.numpy as jnp
from jax import lax
from jax.experimental import pallas as pl
from jax.experimental.pallas import tpu as pltpu
from jax.experimental.pallas import tpu_sc as plsc

N, NP = 10000, 10240
D_IN, H, B, A = 256, 1024, 64, 64
E, ET = 160000, 170000
NS, NC = 16, 2
EPT = 10752
EP = EPT * NS
EB = EPT // 128
CW = 64
NCH = H // CW
CPC = NCH // NC
BM = 1024
RPT = NP // NS


def _mm_body(has_ln, x_ref, w_ref, as_ref, ad_ref, *rest):
    if has_ln:
        b_ref, g_ref, nb_ref, ht_ref, ssrc_ref, sdst_ref, mb_ref, mx_sc = rest
        xb = jnp.concatenate([x_ref[c] for c in range(NCH)], axis=-1)
        xb = xb + b_ref[...]
        mu = jnp.mean(xb, axis=-1, keepdims=True)
        var = jnp.mean((xb - mu) ** 2, axis=-1, keepdims=True)
        xb = (xb - mu) / jnp.sqrt(var + 1e-5) * g_ref[...] + nb_ref[...]
        xb = jnp.maximum(xb, 0.0)
    else:
        ht_ref, ssrc_ref, sdst_ref, mb_ref, mx_sc = rest
        xb = x_ref[...]
    h = jnp.dot(xb, w_ref[...], preferred_element_type=jnp.float32)
    for c in range(NCH):
        ht_ref[c] = h[:, c * CW:(c + 1) * CW]
    s1 = jnp.dot(h, as_ref[...], preferred_element_type=jnp.float32)
    s2 = jnp.dot(h, ad_ref[...], preferred_element_type=jnp.float32)
    ssrc_ref[...] = s1
    sdst_ref[...] = s2

    i = pl.program_id(0)

    @pl.when(i == 0)
    def _():
        mx_sc[0] = -3e38
        mx_sc[1] = -3e38
    mx_sc[0] = jnp.maximum(mx_sc[0], jnp.max(s1))
    mx_sc[1] = jnp.maximum(mx_sc[1], jnp.max(s2))
    msum = mx_sc[0] + mx_sc[1]
    mb_ref[...] = jnp.full((1, 16), jnp.where(msum >= 0.0, msum, 0.2 * msum),
                           jnp.float32)


def _tc_layer(x, W, a_s, a_d, ln_params):
    nblk = NP // BM
    if x.ndim == 3:
        xspec = pl.BlockSpec((NCH, BM, CW), lambda i: (0, i, 0))
        K = H
    else:
        K = x.shape[1]
        xspec = pl.BlockSpec((BM, K), lambda i: (i, 0))
    args = [x, W, a_s.reshape(H, 1), a_d.reshape(H, 1)]
    in_specs = [
        xspec,
        pl.BlockSpec((K, H), lambda i: (0, 0)),
        pl.BlockSpec((H, 1), lambda i: (0, 0)),
        pl.BlockSpec((H, 1), lambda i: (0, 0)),
    ]
    if ln_params is not None:
        bp, g, nb = ln_params
        args += [bp.reshape(1, H), g.reshape(1, H), nb.reshape(1, H)]
        in_specs += [pl.BlockSpec((1, H), lambda i: (0, 0))] * 3
    ht, ssrc, sdst, mb = pl.pallas_call(
        functools.partial(_mm_body, ln_params is not None),
        grid=(nblk,),
        in_specs=in_specs,
        out_specs=[
            pl.BlockSpec((NCH, BM, CW), lambda i: (0, i, 0)),
            pl.BlockSpec((BM, 1), lambda i: (i, 0)),
            pl.BlockSpec((BM, 1), lambda i: (i, 0)),
            pl.BlockSpec((1, 16), lambda i: (0, 0)),
        ],
        out_shape=[
            jax.ShapeDtypeStruct((NCH, NP, CW), jnp.float32),
            jax.ShapeDtypeStruct((NP, 1), jnp.float32),
            jax.ShapeDtypeStruct((NP, 1), jnp.float32),
            jax.ShapeDtypeStruct((1, 16), jnp.float32),
        ],
        scratch_shapes=[pltpu.SMEM((2,), jnp.float32)],
    )(*args)
    return (ht.reshape(NCH * NP, CW), ssrc.reshape(NP), sdst.reshape(NP),
            mb.reshape(16))


def _sc_gat(htab, ssrc, sdst, mb, src3, dst3):
    mesh = plsc.VectorSubcoreMesh(
        core_axis_name="c", subcore_axis_name="s",
        num_cores=NC, num_subcores=NS)

    @functools.partial(
        pl.kernel, mesh=mesh,
        compiler_params=pltpu.CompilerParams(
            needs_layout_passes=False, use_tc_tiling_on_sc=False),
        out_type=jax.ShapeDtypeStruct((NCH, NP, CW), jnp.float32),
        scratch_types=[
            pltpu.VMEM((NP,), jnp.float32),
            pltpu.VMEM((NP,), jnp.float32),
            pltpu.VMEM((NP,), jnp.float32),
            pltpu.VMEM((EB, 128), jnp.int32),
            pltpu.VMEM((EB, 128), jnp.int32),
            pltpu.VMEM((EB, 128), jnp.float32),
            pltpu.VMEM((2, 128), jnp.int32),
            pltpu.VMEM((2, 128, CW), jnp.float32),
            pltpu.VMEM((128, CW), jnp.float32),
            pltpu.VMEM((RPT,), jnp.float32),
            pltpu.VMEM((RPT,), jnp.float32),
            pltpu.HBM((NC, NS, NP), jnp.float32),
            pltpu.VMEM_SHARED((NP,), jnp.float32),
            pltpu.VMEM_SHARED((NP, CW), jnp.float32),
            pltpu.VMEM((16,), jnp.float32),
            pltpu.SemaphoreType.DMA,
            pltpu.SemaphoreType.DMA,
        ],
    )
    def k(htab_h, ssrc_h, sdst_h, mb_h, src_h, dst_h, agg_h,
          ssrc_t, sdst_t, s_t, src_t, dst_t, alpha_t, idx2, rowbuf2, zbuf,
          tmp_t, acc_t, s_all, s_acc, out_chunk, mb_t, gsem, ssem):
        cid = lax.axis_index("c")
        sid = lax.axis_index("s")
        pltpu.sync_copy(ssrc_h, ssrc_t)
        pltpu.sync_copy(sdst_h, sdst_t)
        pltpu.sync_copy(mb_h, mb_t)
        pltpu.sync_copy(src_h.at[sid], src_t)
        pltpu.sync_copy(dst_h.at[sid], dst_t)

        zf = jnp.zeros((16,), jnp.float32)

        def zero_s(i, c):
            s_t[pl.ds(i * 16, 16)] = zf
            return c
        lax.fori_loop(0, NP // 16, zero_s, 0)

        def zero_z(r, c):
            for q in range(CW // 16):
                zbuf[r, pl.ds(q * 16, 16)] = zf
            return c
        lax.fori_loop(0, 128, zero_z, 0)

        mb = mb_t[pl.ds(0, 16)]

        ebase = sid * EPT
        iota16 = lax.iota(jnp.int32, 16)

        def e_body(b, c):
            for kq in range(8):
                sl = pl.ds(kq * 16, 16)
                sv = src_t[b, sl]
                dv = dst_t[b, sl]
                e = (plsc.load_gather(ssrc_t, [sv])
                     + plsc.load_gather(sdst_t, [dv]))
                e = jnp.where(e >= 0.0, e, 0.2 * e) - mb
                ex = jnp.exp(e)
                gidx = ebase + b * 128 + kq * 16 + iota16
                ex = jnp.where(gidx < ET, ex, 0.0)
                alpha_t[b, sl] = ex
                plsc.addupdate_scatter(s_t, [dv], ex)
            return c
        lax.fori_loop(0, EB, e_body, 0)

        plsc.subcore_barrier()
        pltpu.sync_copy(s_t, s_all.at[cid, sid])
        plsc.subcore_barrier()
        base = sid * RPT
        pltpu.sync_copy(s_all.at[cid, 0, pl.ds(base, RPT)], acc_t)

        def t_body(t, c):
            pltpu.sync_copy(s_all.at[cid, t, pl.ds(base, RPT)], tmp_t)

            def add_body(j, c2):
                sl = pl.ds(j * 16, 16)
                acc_t[sl] = acc_t[sl] + tmp_t[sl]
                return c2
            lax.fori_loop(0, RPT // 16, add_body, 0)
            return c
        lax.fori_loop(1, NS, t_body, 0)
        pltpu.sync_copy(acc_t, s_acc.at[pl.ds(base, RPT)])
        plsc.subcore_barrier()
        pltpu.sync_copy(s_acc, s_t)

        def a_body(b, c):
            for kq in range(8):
                sl = pl.ds(kq * 16, 16)
                dv = dst_t[b, sl]
                sden = plsc.load_gather(s_t, [dv])
                alpha_t[b, sl] = alpha_t[b, sl] / sden
            return c
        lax.fori_loop(0, EB, a_body, 0)

        for cc in range(CPC):
            g = cid * CPC + cc
            goff = g * NP
            plsc.subcore_barrier()
            for z in range(RPT // 128):
                pltpu.sync_copy(zbuf, out_chunk.at[pl.ds(sid * RPT + z * 128, 128), :])
            plsc.subcore_barrier()

            def build_idx(q, b):
                for kq in range(8):
                    sl = pl.ds(kq * 16, 16)
                    idx2[q, sl] = src_t[b, sl] + goff

            build_idx(0, 0)
            pltpu.async_copy(htab_h.at[idx2.at[0]], rowbuf2.at[0], gsem)

            def grp(b, c):
                p = b % 2

                @pl.when(b >= 1)
                def _():
                    pltpu.make_async_copy(
                        rowbuf2.at[1 - p], out_chunk.at[dst_t.at[b - 1]],
                        ssem).wait()

                @pl.when(b + 1 < EB)
                def _():
                    build_idx(1 - p, b + 1)
                    pltpu.async_copy(htab_h.at[idx2.at[1 - p]],
                                     rowbuf2.at[1 - p], gsem)

                pltpu.make_async_copy(htab_h.at[idx2.at[p]], rowbuf2.at[p],
                                      gsem).wait()

                @plsc.parallel_loop(0, 8, unroll=4)
                def _(kq2):
                    av16 = alpha_t[b, pl.ds(kq2 * 16, 16)]
                    for u in range(16):
                        av = jnp.full((16,), av16[u], jnp.float32)
                        r = kq2 * 16 + u
                        for q in range(CW // 16):
                            ql = pl.ds(q * 16, 16)
                            rowbuf2[p, r, ql] = rowbuf2[p, r, ql] * av

                pltpu.async_copy(rowbuf2.at[p], out_chunk.at[dst_t.at[b]],
                                 ssem, add=True)
                return c
            lax.fori_loop(0, EB, grp, 0)

            pltpu.make_async_copy(rowbuf2.at[(EB - 1) % 2],
                                  out_chunk.at[dst_t.at[EB - 1]], ssem).wait()

            plsc.subcore_barrier()
            for z in range(RPT // 128):
                r0 = sid * RPT + z * 128
                pltpu.sync_copy(out_chunk.at[pl.ds(r0, 128), :],
                                agg_h.at[g, pl.ds(r0, 128), :])

    return k(htab, ssrc, sdst, mb, src3, dst3)


def _pool_body(x_ref, b_ref, g_ref, nb_ref, bat_ref, wg_ref, bg_ref,
               pn_ref, sp_ref):
    i = pl.program_id(0)
    xb = jnp.concatenate([x_ref[c] for c in range(NCH)], axis=-1)
    xb = xb + b_ref[...]
    mu = jnp.mean(xb, axis=-1, keepdims=True)
    var = jnp.mean((xb - mu) ** 2, axis=-1, keepdims=True)
    xb = (xb - mu) / jnp.sqrt(var + 1e-5) * g_ref[...] + nb_ref[...]
    xb = jnp.maximum(xb, 0.0)
    gl = jnp.dot(xb, wg_ref[...], preferred_element_type=jnp.float32) + bg_ref[...]
    ex = jnp.exp(jax.nn.sigmoid(gl))
    onehot = (bat_ref[...] == lax.broadcasted_iota(jnp.int32, (BM, B), 1)
              ).astype(jnp.float32)
    w = onehot * ex
    pp = lax.dot_general(w, xb, (((0,), (0,)), ((), ())),
                         preferred_element_type=jnp.float32)
    sp1 = lax.dot_general(w, jnp.ones((BM, 1), jnp.float32),
                          (((0,), (0,)), ((), ())),
                          preferred_element_type=jnp.float32)

    @pl.when(i == 0)
    def _():
        pn_ref[...] = jnp.zeros_like(pn_ref)
        sp_ref[...] = jnp.zeros_like(sp_ref)
    pn_ref[...] += pp
    sp_ref[...] += sp1


def _tc_pool(x, ln_params, bat2, Wg, bg):
    bp, g, nb = ln_params
    nblk = NP // BM
    return pl.pallas_call(
        _pool_body,
        grid=(nblk,),
        in_specs=[
            pl.BlockSpec((NCH, BM, CW), lambda i: (0, i, 0)),
            pl.BlockSpec((1, H), lambda i: (0, 0)),
            pl.BlockSpec((1, H), lambda i: (0, 0)),
            pl.BlockSpec((1, H), lambda i: (0, 0)),
            pl.BlockSpec((BM, 1), lambda i: (i, 0)),
            pl.BlockSpec((H, 1), lambda i: (0, 0)),
            pl.BlockSpec((1, 1), lambda i: (0, 0)),
        ],
        out_specs=[
            pl.BlockSpec((B, H), lambda i: (0, 0)),
            pl.BlockSpec((B, 1), lambda i: (0, 0)),
        ],
        out_shape=[
            jax.ShapeDtypeStruct((B, H), jnp.float32),
            jax.ShapeDtypeStruct((B, 1), jnp.float32),
        ],
    )(x, bp.reshape(1, H), g.reshape(1, H), nb.reshape(1, H),
      bat2, Wg, bg.reshape(1, 1))


def _mlp_body(pn_ref, sp_ref, *refs):
    y = pn_ref[...] / (sp_ref[...] + 1e-16)
    for l in range(5):
        fw, fb, fg, fbeta = refs[4 * l:4 * l + 4]
        y = jnp.dot(y, fw[...], preferred_element_type=jnp.float32) + fb[...]
        mu = jnp.mean(y, axis=-1, keepdims=True)
        var = jnp.mean((y - mu) ** 2, axis=-1, keepdims=True)
        y = (y - mu) / jnp.sqrt(var + 1e-5) * fg[...] + fbeta[...]
        y = jnp.maximum(y, 0.0)
    wo, bo, out_ref = refs[20], refs[21], refs[22]
    out_ref[...] = (jnp.dot(y, wo[...], preferred_element_type=jnp.float32)
                    + bo[...])


def _tc_mlp(pn, sp, params):
    args = [pn, sp]
    for l in range(5):
        args += [params[f"fW{l}"], params[f"fb{l}"].reshape(1, H),
                 params[f"fg{l}"].reshape(1, H), params[f"fbeta{l}"].reshape(1, H)]
    args += [params["Wo"], params["bo"].reshape(1, A)]
    return pl.pallas_call(
        _mlp_body,
        out_shape=jax.ShapeDtypeStruct((B, A), jnp.float32),
    )(*args)


def kernel(tree_x, edge_index, batch, params):
    idt = edge_index.dtype
    loops = jnp.arange(N, dtype=idt)
    pad = jnp.zeros((EP - ET,), idt)
    src3 = jnp.concatenate([edge_index[0], loops, pad]).reshape(NS, EB, 128)
    dst3 = jnp.concatenate([edge_index[1], loops, pad]).reshape(NS, EB, 128)
    x = jnp.zeros((NP, D_IN), jnp.float32).at[:N].set(tree_x)
    bat2 = jnp.full((NP, 1), B, jnp.int32).at[:N, 0].set(batch)

    ln = None
    for i in range(5):
        ht, ssrc, sdst, mb = _tc_layer(
            x, params[f"W{i}"], params[f"as{i}"], params[f"ad{i}"], ln)
        x = _sc_gat(ht, ssrc, sdst, mb, src3, dst3)
        ln = (params[f"b{i}"], params[f"ng{i}"], params[f"nb{i}"])

    pn, sp = _tc_pool(x, ln, bat2, params["Wg"], params["bg"])
    return _tc_mlp(pn, sp, params)


# speedup vs baseline: 3.0119x; 1.1474x over previous
import functools

import jax
import jax.numpy as jnp
from jax import lax
from jax.experimental import pallas as pl
from jax.experimental.pallas import tpu as pltpu
from jax.experimental.pallas import tpu_sc as plsc

N, NP = 10000, 10240
D_IN, H, B, A = 256, 1024, 64, 64
E, ET = 160000, 170000
NS, NC = 16, 2
EPT = 10752
EP = EPT * NS
EB = EPT // 128
CW = 64
NCH = H // CW
CPC = NCH // NC
BM = 1024
RPT = NP // NS


def _mm_body(has_ln, x_ref, w_ref, as_ref, ad_ref, *rest):
    if has_ln:
        b_ref, g_ref, nb_ref, ht_ref, ssrc_ref, sdst_ref, mb_ref, mx_sc = rest
        xb = jnp.concatenate([x_ref[c] for c in range(NCH)], axis=-1)
        xb = xb + b_ref[...]
        mu = jnp.mean(xb, axis=-1, keepdims=True)
        var = jnp.mean((xb - mu) ** 2, axis=-1, keepdims=True)
        xb = (xb - mu) / jnp.sqrt(var + 1e-5) * g_ref[...] + nb_ref[...]
        xb = jnp.maximum(xb, 0.0)
    else:
        ht_ref, ssrc_ref, sdst_ref, mb_ref, mx_sc = rest
        xb = x_ref[...]
    h = jnp.dot(xb, w_ref[...], preferred_element_type=jnp.float32)
    for c in range(NCH):
        ht_ref[c] = h[:, c * CW:(c + 1) * CW]
    s1 = jnp.dot(h, as_ref[...], preferred_element_type=jnp.float32)
    s2 = jnp.dot(h, ad_ref[...], preferred_element_type=jnp.float32)
    ssrc_ref[...] = s1
    sdst_ref[...] = s2

    i = pl.program_id(0)

    @pl.when(i == 0)
    def _():
        mx_sc[0] = -3e38
        mx_sc[1] = -3e38
    mx_sc[0] = jnp.maximum(mx_sc[0], jnp.max(s1))
    mx_sc[1] = jnp.maximum(mx_sc[1], jnp.max(s2))
    msum = mx_sc[0] + mx_sc[1]
    mb_ref[...] = jnp.full((1, 16), jnp.where(msum >= 0.0, msum, 0.2 * msum),
                           jnp.float32)


def _tc_layer(x, W, a_s, a_d, ln_params):
    nblk = NP // BM
    if x.ndim == 3:
        xspec = pl.BlockSpec((NCH, BM, CW), lambda i: (0, i, 0))
        K = H
    else:
        K = x.shape[1]
        xspec = pl.BlockSpec((BM, K), lambda i: (i, 0))
    args = [x, W, a_s.reshape(H, 1), a_d.reshape(H, 1)]
    in_specs = [
        xspec,
        pl.BlockSpec((K, H), lambda i: (0, 0)),
        pl.BlockSpec((H, 1), lambda i: (0, 0)),
        pl.BlockSpec((H, 1), lambda i: (0, 0)),
    ]
    if ln_params is not None:
        bp, g, nb = ln_params
        args += [bp.reshape(1, H), g.reshape(1, H), nb.reshape(1, H)]
        in_specs += [pl.BlockSpec((1, H), lambda i: (0, 0))] * 3
    ht, ssrc, sdst, mb = pl.pallas_call(
        functools.partial(_mm_body, ln_params is not None),
        grid=(nblk,),
        in_specs=in_specs,
        out_specs=[
            pl.BlockSpec((NCH, BM, CW), lambda i: (0, i, 0)),
            pl.BlockSpec((BM, 1), lambda i: (i, 0)),
            pl.BlockSpec((BM, 1), lambda i: (i, 0)),
            pl.BlockSpec((1, 16), lambda i: (0, 0)),
        ],
        out_shape=[
            jax.ShapeDtypeStruct((NCH, NP, CW), jnp.float32),
            jax.ShapeDtypeStruct((NP, 1), jnp.float32),
            jax.ShapeDtypeStruct((NP, 1), jnp.float32),
            jax.ShapeDtypeStruct((1, 16), jnp.float32),
        ],
        scratch_shapes=[pltpu.SMEM((2,), jnp.float32)],
    )(*args)
    return (ht.reshape(NCH * NP, CW), ssrc.reshape(NP), sdst.reshape(NP),
            mb.reshape(16))


def _sc_gat(htab, ssrc, sdst, mb, src3, dst3):
    mesh = plsc.VectorSubcoreMesh(
        core_axis_name="c", subcore_axis_name="s",
        num_cores=NC, num_subcores=NS)

    @functools.partial(
        pl.kernel, mesh=mesh,
        compiler_params=pltpu.CompilerParams(
            needs_layout_passes=False, use_tc_tiling_on_sc=False),
        out_type=jax.ShapeDtypeStruct((NCH, NP, CW), jnp.float32),
        scratch_types=[
            pltpu.VMEM((NP,), jnp.float32),
            pltpu.VMEM((NP,), jnp.float32),
            pltpu.VMEM((NP,), jnp.float32),
            pltpu.VMEM((EB, 128), jnp.int32),
            pltpu.VMEM((EB, 128), jnp.int32),
            pltpu.VMEM((EB, 128), jnp.float32),
            pltpu.VMEM((1, 128), jnp.int32),
            pltpu.VMEM((1, 128), jnp.int32),
            pltpu.VMEM((1, 128), jnp.int32),
            pltpu.VMEM((128, CW), jnp.float32),
            pltpu.VMEM((128, CW), jnp.float32),
            pltpu.VMEM((128, CW), jnp.float32),
            pltpu.VMEM((RPT,), jnp.float32),
            pltpu.VMEM((RPT,), jnp.float32),
            pltpu.HBM((NC, NS, NP), jnp.float32),
            pltpu.VMEM_SHARED((NP,), jnp.float32),
            pltpu.VMEM_SHARED((NP, CW), jnp.float32),
            pltpu.VMEM((16,), jnp.float32),
            pltpu.SemaphoreType.DMA,
            pltpu.SemaphoreType.DMA,
        ],
    )
    def k(htab_h, ssrc_h, sdst_h, mb_h, src_h, dst_h, agg_h,
          ssrc_t, sdst_t, s_t, src_t, dst_t, alpha_t, ixA, ixB, ixC,
          rbA, rbB, rbC,
          tmp_t, acc_t, s_all, s_acc, out_chunk, mb_t, gsem, ssem):
        cid = lax.axis_index("c")
        sid = lax.axis_index("s")
        pltpu.sync_copy(ssrc_h, ssrc_t)
        pltpu.sync_copy(sdst_h, sdst_t)
        pltpu.sync_copy(mb_h, mb_t)
        pltpu.sync_copy(src_h.at[sid], src_t)
        pltpu.sync_copy(dst_h.at[sid], dst_t)

        zf = jnp.zeros((16,), jnp.float32)

        def zero_s(i, c):
            s_t[pl.ds(i * 16, 16)] = zf
            return c
        lax.fori_loop(0, NP // 16, zero_s, 0)

        mb = mb_t[pl.ds(0, 16)]

        ebase = sid * EPT
        iota16 = lax.iota(jnp.int32, 16)

        def e_body(b, c):
            for kq in range(8):
                sl = pl.ds(kq * 16, 16)
                sv = src_t[b, sl]
                dv = dst_t[b, sl]
                e = (plsc.load_gather(ssrc_t, [sv])
                     + plsc.load_gather(sdst_t, [dv]))
                e = jnp.where(e >= 0.0, e, 0.2 * e) - mb
                ex = jnp.exp(e)
                gidx = ebase + b * 128 + kq * 16 + iota16
                ex = jnp.where(gidx < ET, ex, 0.0)
                alpha_t[b, sl] = ex
                plsc.addupdate_scatter(s_t, [dv], ex)
            return c
        lax.fori_loop(0, EB, e_body, 0)

        plsc.subcore_barrier()
        pltpu.sync_copy(s_t, s_all.at[cid, sid])
        plsc.subcore_barrier()
        base = sid * RPT
        pltpu.sync_copy(s_all.at[cid, 0, pl.ds(base, RPT)], acc_t)

        def t_body(t, c):
            pltpu.sync_copy(s_all.at[cid, t, pl.ds(base, RPT)], tmp_t)

            def add_body(j, c2):
                sl = pl.ds(j * 16, 16)
                acc_t[sl] = acc_t[sl] + tmp_t[sl]
                return c2
            lax.fori_loop(0, RPT // 16, add_body, 0)
            return c
        lax.fori_loop(1, NS, t_body, 0)
        pltpu.sync_copy(acc_t, s_acc.at[pl.ds(base, RPT)])
        plsc.subcore_barrier()
        pltpu.sync_copy(s_acc, s_t)

        def a_body(b, c):
            for kq in range(8):
                sl = pl.ds(kq * 16, 16)
                dv = dst_t[b, sl]
                sden = plsc.load_gather(s_t, [dv])
                alpha_t[b, sl] = alpha_t[b, sl] / sden
            return c
        lax.fori_loop(0, EB, a_body, 0)

        for cc in range(CPC):
            g = cid * CPC + cc
            goff = g * NP
            plsc.subcore_barrier()

            def zero_rb(r, c):
                for q in range(CW // 16):
                    rbA[r, pl.ds(q * 16, 16)] = zf
                return c
            lax.fori_loop(0, 128, zero_rb, 0)
            for z in range(RPT // 128):
                pltpu.sync_copy(rbA, out_chunk.at[pl.ds(sid * RPT + z * 128, 128), :])
            plsc.subcore_barrier()

            bufs = ((rbA, ixA), (rbB, ixB), (rbC, ixC))

            def fire_gather(i, b):
                rb, ix = bufs[i]
                for kq in range(8):
                    sl = pl.ds(kq * 16, 16)
                    ix[0, sl] = src_t[b, sl] + goff
                pltpu.async_copy(htab_h.at[ix.at[0]], rb, gsem)

            def wait_gather(i):
                rb, ix = bufs[i]
                pltpu.make_async_copy(htab_h.at[ix.at[0]], rb, gsem).wait()

            def scale_buf(i, b):
                rb = bufs[i][0]

                @plsc.parallel_loop(0, 8, unroll=2)
                def _(kq2):
                    av16 = alpha_t[b, pl.ds(kq2 * 16, 16)]
                    for u in range(16):
                        av = jnp.full((16,), av16[u], jnp.float32)
                        r = kq2 * 16 + u
                        for q in range(CW // 16):
                            ql = pl.ds(q * 16, 16)
                            rb[r, ql] = rb[r, ql] * av

            def fire_scatter(i, b):
                pltpu.async_copy(bufs[i][0], out_chunk.at[dst_t.at[b]],
                                 ssem, add=True)

            def drain_scatter(i, b):
                pltpu.make_async_copy(bufs[i][0], out_chunk.at[dst_t.at[b]],
                                      ssem).wait()

            def dispatch(sel, fn):
                for i in range(3):
                    @pl.when(sel == i)
                    def _go(i=i):
                        fn(i)

            fire_gather(0, 0)

            def grp(b, c):
                p = b % 3
                q = (b + 1) % 3

                @pl.when(b >= 2)
                def _():
                    dispatch(q, lambda i: drain_scatter(i, b - 2))

                @pl.when(b + 1 < EB)
                def _():
                    dispatch(q, lambda i: fire_gather(i, b + 1))

                dispatch(p, wait_gather)
                dispatch(p, lambda i: scale_buf(i, b))
                dispatch(p, lambda i: fire_scatter(i, b))
                return c
            lax.fori_loop(0, EB, grp, 0)

            drain_scatter((EB - 2) % 3, EB - 2)
            drain_scatter((EB - 1) % 3, EB - 1)

            plsc.subcore_barrier()
            for z in range(RPT // 128):
                r0 = sid * RPT + z * 128
                pltpu.sync_copy(out_chunk.at[pl.ds(r0, 128), :],
                                agg_h.at[g, pl.ds(r0, 128), :])

    return k(htab, ssrc, sdst, mb, src3, dst3)


def _pool_body(x_ref, b_ref, g_ref, nb_ref, bat_ref, wg_ref, bg_ref,
               pn_ref, sp_ref):
    i = pl.program_id(0)
    xb = jnp.concatenate([x_ref[c] for c in range(NCH)], axis=-1)
    xb = xb + b_ref[...]
    mu = jnp.mean(xb, axis=-1, keepdims=True)
    var = jnp.mean((xb - mu) ** 2, axis=-1, keepdims=True)
    xb = (xb - mu) / jnp.sqrt(var + 1e-5) * g_ref[...] + nb_ref[...]
    xb = jnp.maximum(xb, 0.0)
    gl = jnp.dot(xb, wg_ref[...], preferred_element_type=jnp.float32) + bg_ref[...]
    ex = jnp.exp(jax.nn.sigmoid(gl))
    onehot = (bat_ref[...] == lax.broadcasted_iota(jnp.int32, (BM, B), 1)
              ).astype(jnp.float32)
    w = onehot * ex
    pp = lax.dot_general(w, xb, (((0,), (0,)), ((), ())),
                         preferred_element_type=jnp.float32)
    sp1 = lax.dot_general(w, jnp.ones((BM, 1), jnp.float32),
                          (((0,), (0,)), ((), ())),
                          preferred_element_type=jnp.float32)

    @pl.when(i == 0)
    def _():
        pn_ref[...] = jnp.zeros_like(pn_ref)
        sp_ref[...] = jnp.zeros_like(sp_ref)
    pn_ref[...] += pp
    sp_ref[...] += sp1


def _tc_pool(x, ln_params, bat2, Wg, bg):
    bp, g, nb = ln_params
    nblk = NP // BM
    return pl.pallas_call(
        _pool_body,
        grid=(nblk,),
        in_specs=[
            pl.BlockSpec((NCH, BM, CW), lambda i: (0, i, 0)),
            pl.BlockSpec((1, H), lambda i: (0, 0)),
            pl.BlockSpec((1, H), lambda i: (0, 0)),
            pl.BlockSpec((1, H), lambda i: (0, 0)),
            pl.BlockSpec((BM, 1), lambda i: (i, 0)),
            pl.BlockSpec((H, 1), lambda i: (0, 0)),
            pl.BlockSpec((1, 1), lambda i: (0, 0)),
        ],
        out_specs=[
            pl.BlockSpec((B, H), lambda i: (0, 0)),
            pl.BlockSpec((B, 1), lambda i: (0, 0)),
        ],
        out_shape=[
            jax.ShapeDtypeStruct((B, H), jnp.float32),
            jax.ShapeDtypeStruct((B, 1), jnp.float32),
        ],
    )(x, bp.reshape(1, H), g.reshape(1, H), nb.reshape(1, H),
      bat2, Wg, bg.reshape(1, 1))


def _mlp_body(pn_ref, sp_ref, *refs):
    y = pn_ref[...] / (sp_ref[...] + 1e-16)
    for l in range(5):
        fw, fb, fg, fbeta = refs[4 * l:4 * l + 4]
        y = jnp.dot(y, fw[...], preferred_element_type=jnp.float32) + fb[...]
        mu = jnp.mean(y, axis=-1, keepdims=True)
        var = jnp.mean((y - mu) ** 2, axis=-1, keepdims=True)
        y = (y - mu) / jnp.sqrt(var + 1e-5) * fg[...] + fbeta[...]
        y = jnp.maximum(y, 0.0)
    wo, bo, out_ref = refs[20], refs[21], refs[22]
    out_ref[...] = (jnp.dot(y, wo[...], preferred_element_type=jnp.float32)
                    + bo[...])


def _tc_mlp(pn, sp, params):
    args = [pn, sp]
    for l in range(5):
        args += [params[f"fW{l}"], params[f"fb{l}"].reshape(1, H),
                 params[f"fg{l}"].reshape(1, H), params[f"fbeta{l}"].reshape(1, H)]
    args += [params["Wo"], params["bo"].reshape(1, A)]
    return pl.pallas_call(
        _mlp_body,
        out_shape=jax.ShapeDtypeStruct((B, A), jnp.float32),
    )(*args)


def kernel(tree_x, edge_index, batch, params):
    idt = edge_index.dtype
    loops = jnp.arange(N, dtype=idt)
    pad = jnp.zeros((EP - ET,), idt)
    src3 = jnp.concatenate([edge_index[0], loops, pad]).reshape(NS, EB, 128)
    dst3 = jnp.concatenate([edge_index[1], loops, pad]).reshape(NS, EB, 128)
    x = jnp.zeros((NP, D_IN), jnp.float32).at[:N].set(tree_x)
    bat2 = jnp.full((NP, 1), B, jnp.int32).at[:N, 0].set(batch)

    ln = None
    for i in range(5):
        ht, ssrc, sdst, mb = _tc_layer(
            x, params[f"W{i}"], params[f"as{i}"], params[f"ad{i}"], ln)
        x = _sc_gat(ht, ssrc, sdst, mb, src3, dst3)
        ln = (params[f"b{i}"], params[f"ng{i}"], params[f"nb{i}"])

    pn, sp = _tc_pool(x, ln, bat2, params["Wg"], params["bg"])
    return _tc_mlp(pn, sp, params)
